# Initial kernel scaffold; baseline (speedup 1.0000x reference)
import jax
import jax.numpy as jnp
from jax.experimental import pallas as pl


def kernel(x, conv_w, conv_b, bn1_g, bn1_b, bn2_g, bn2_b):
    raise NotImplementedError("write your pallas kernel here")


# speedup vs baseline: 2.5699x; 2.5699x over previous
import functools

import jax
import jax.numpy as jnp
from jax.experimental import pallas as pl
from jax.experimental.pallas import tpu as pltpu

EPS = 1e-5


def _pick_tile(n, target=512):
    if n <= target:
        return n
    t = (target // 128) * 128
    while t >= 128:
        if n % t == 0:
            return t
        t -= 128
    return n


def _xstats_kernel(x_ref, sum_ref, sq_ref):
    xb = x_ref[0]
    sum_ref[0] = jnp.sum(xb, axis=-1, keepdims=True)
    sq_ref[0] = jnp.sum(xb * xb, axis=-1, keepdims=True)


def _corr_kernel(x_ref, s1_ref, t1_ref, d_ref, hsum_ref, edge_ref, *, C, L):
    h = jnp.maximum(x_ref[0] * s1_ref[...] + t1_ref[...], 0.0)
    hs = jnp.concatenate([h[:, 1:], jnp.zeros((C, 1), jnp.float32)], axis=1)
    hb = h.astype(jnp.bfloat16)
    rhs = jnp.concatenate([hb, hs.astype(jnp.bfloat16)], axis=0)
    d_ref[0] = jax.lax.dot_general(
        hb, rhs, (((1,), (1,)), ((), ())),
        preferred_element_type=jnp.float32)
    hsum_ref[0] = jnp.sum(h, axis=-1, keepdims=True)
    edge_ref[0] = jnp.concatenate([h[:, 0:1], h[:, L - 1:L]], axis=1)


def _out_kernel(x_ref, s1_ref, t1_ref, we_ref, wo_ref, s2_ref, t2_ref,
                o_ref, hb3_ref, *, C, L, TL):
    t = pl.program_id(1)

    @pl.when(t == 0)
    def _():
        h = jnp.maximum(x_ref[0] * s1_ref[...] + t1_ref[...], 0.0)
        z1 = jnp.zeros((C, 1), jnp.float32)
        hm = jnp.concatenate([z1, h[:, :L - 1]], axis=1)
        hp = jnp.concatenate([h[:, 1:], z1], axis=1)
        hb3_ref[0:C, :] = hm.astype(jnp.bfloat16)
        hb3_ref[C:2 * C, :] = h.astype(jnp.bfloat16)
        hb3_ref[2 * C:3 * C, :] = hp.astype(jnp.bfloat16)

    T0 = t * TL
    sA = hb3_ref[0:2 * C, pl.ds(T0, TL)]
    sB = hb3_ref[C:3 * C, pl.ds(T0, TL)]
    yE = jnp.dot(we_ref[...], sA, preferred_element_type=jnp.float32)
    yO = jnp.dot(wo_ref[...], sB, preferred_element_type=jnp.float32)
    s2 = s2_ref[...]
    t2 = t2_ref[...]
    zE = jnp.maximum(yE * s2 + t2, 0.0)
    zO = jnp.maximum(yO * s2 + t2, 0.0)
    F = zE.shape[0]
    iota = jax.lax.broadcasted_iota(jnp.int32, (F, 128), 1)
    idx = (iota >> 1) + ((iota & 1) << 6)
    for s in range(2 * TL // 128):
        src = jnp.concatenate(
            [zE[:, 64 * s:64 * s + 64], zO[:, 64 * s:64 * s + 64]], axis=1)
        o_ref[0, :, pl.ds(128 * s, 128)] = jnp.take_along_axis(src, idx,
                                                               axis=1)


def kernel(x, conv_w, conv_b, bn1_g, bn1_b, bn2_g, bn2_b):
    N, C, L = x.shape
    F = conv_w.shape[0]
    L2 = 2 * L
    TL = _pick_tile(L)
    T = L // TL

    x = x.astype(jnp.float32)
    conv_w = conv_w.astype(jnp.float32)
    conv_b = conv_b.astype(jnp.float32)
    bn1_g = bn1_g.astype(jnp.float32)
    bn1_b = bn1_b.astype(jnp.float32)
    bn2_g = bn2_g.astype(jnp.float32)
    bn2_b = bn2_b.astype(jnp.float32)

    vmem_limit = 64 * 1024 * 1024
    cp1 = pltpu.CompilerParams(dimension_semantics=("parallel",),
                               vmem_limit_bytes=vmem_limit)
    cp2 = pltpu.CompilerParams(dimension_semantics=("parallel", "arbitrary"),
                               vmem_limit_bytes=vmem_limit)

    xsum, xsq = pl.pallas_call(
        _xstats_kernel,
        out_shape=(jax.ShapeDtypeStruct((N, C, 1), jnp.float32),
                   jax.ShapeDtypeStruct((N, C, 1), jnp.float32)),
        grid=(N,),
        in_specs=[pl.BlockSpec((1, C, L), lambda n: (n, 0, 0))],
        out_specs=(pl.BlockSpec((1, C, 1), lambda n: (n, 0, 0)),
                   pl.BlockSpec((1, C, 1), lambda n: (n, 0, 0))),
        compiler_params=cp1,
    )(x)

    cnt1 = float(N * L)
    mean1 = jnp.sum(xsum, axis=0)[:, 0] / cnt1
    var1 = jnp.sum(xsq, axis=0)[:, 0] / cnt1 - mean1 ** 2
    sc1 = bn1_g * jax.lax.rsqrt(var1 + EPS)
    s1 = sc1.reshape(C, 1)
    t1 = (bn1_b - mean1 * sc1).reshape(C, 1)

    d, hsum, edge = pl.pallas_call(
        functools.partial(_corr_kernel, C=C, L=L),
        out_shape=(jax.ShapeDtypeStruct((N, C, 2 * C), jnp.float32),
                   jax.ShapeDtypeStruct((N, C, 1), jnp.float32),
                   jax.ShapeDtypeStruct((N, C, 2), jnp.float32)),
        grid=(N,),
        in_specs=[pl.BlockSpec((1, C, L), lambda n: (n, 0, 0)),
                  pl.BlockSpec((C, 1), lambda n: (0, 0)),
                  pl.BlockSpec((C, 1), lambda n: (0, 0))],
        out_specs=(pl.BlockSpec((1, C, 2 * C), lambda n: (n, 0, 0)),
                   pl.BlockSpec((1, C, 1), lambda n: (n, 0, 0)),
                   pl.BlockSpec((1, C, 2), lambda n: (n, 0, 0))),
        compiler_params=cp1,
    )(x, s1, t1)

    W0 = conv_w[:, :, 0]
    W1 = conv_w[:, :, 1]
    W2 = conv_w[:, :, 2]
    A = W0
    Bv = W1 + W2
    Cm = W0 + W1
    Dv = W2

    D0 = jnp.sum(d[:, :, :C], axis=0)
    D1 = jnp.sum(d[:, :, C:], axis=0)
    Sh = jnp.sum(hsum[:, :, 0], axis=0)
    H0 = edge[:, :, 0]
    HL = edge[:, :, 1]
    Q0 = H0.T @ H0
    QL = HL.T @ HL
    S0 = jnp.sum(H0, axis=0)
    SL = jnp.sum(HL, axis=0)

    def rs(M, W):
        return jnp.sum(M * W, axis=1)

    linE = A @ (Sh - SL) + Bv @ Sh
    linO = Cm @ Sh + Dv @ (Sh - S0)
    lin = linE + linO
    quad = (rs(A @ (D0 - QL), A) + rs(Bv @ D0, Bv) + 2.0 * rs(A @ D1, Bv)
            + rs(Cm @ D0, Cm) + rs(Dv @ (D0 - Q0), Dv)
            + 2.0 * rs(Cm @ D1, Dv))
    cnt2 = float(N * L2)
    ysum = lin + cnt2 * conv_b
    ysq = quad + 2.0 * conv_b * lin + cnt2 * conv_b ** 2
    mean2 = ysum / cnt2
    var2 = ysq / cnt2 - mean2 ** 2
    sc2 = bn2_g * jax.lax.rsqrt(var2 + EPS)
    s2 = sc2.reshape(F, 1)
    t2 = (bn2_b - mean2 * sc2 + sc2 * conv_b).reshape(F, 1)

    we = jnp.concatenate([A, Bv], axis=1).astype(jnp.bfloat16)
    wo = jnp.concatenate([Cm, Dv], axis=1).astype(jnp.bfloat16)

    z = pl.pallas_call(
        functools.partial(_out_kernel, C=C, L=L, TL=TL),
        out_shape=jax.ShapeDtypeStruct((N, F, L2), jnp.float32),
        grid=(N, T),
        in_specs=[pl.BlockSpec((1, C, L), lambda n, t: (n, 0, 0)),
                  pl.BlockSpec((C, 1), lambda n, t: (0, 0)),
                  pl.BlockSpec((C, 1), lambda n, t: (0, 0)),
                  pl.BlockSpec((F, 2 * C), lambda n, t: (0, 0)),
                  pl.BlockSpec((F, 2 * C), lambda n, t: (0, 0)),
                  pl.BlockSpec((F, 1), lambda n, t: (0, 0)),
                  pl.BlockSpec((F, 1), lambda n, t: (0, 0))],
        out_specs=pl.BlockSpec((1, F, 2 * TL), lambda n, t: (n, 0, t)),
        scratch_shapes=[pltpu.VMEM((3 * C, L), jnp.bfloat16)],
        compiler_params=cp2,
    )(x, s1, t1, we, wo, s2, t2)
    return z


# speedup vs baseline: 3.1098x; 1.2101x over previous
import functools

import jax
import jax.numpy as jnp
from jax.experimental import pallas as pl
from jax.experimental.pallas import tpu as pltpu

EPS = 1e-5


def _pick_tile(n, target):
    if n <= target:
        return n
    t = (target // 128) * 128
    while t >= 128:
        if n % t == 0:
            return t
        t -= 128
    return n


def _xstats_kernel(x_ref, sum_ref, sq_ref):
    n = pl.program_id(0)

    @pl.when(n == 0)
    def _():
        sum_ref[...] = jnp.zeros_like(sum_ref)
        sq_ref[...] = jnp.zeros_like(sq_ref)

    xb = x_ref[0]
    sum_ref[...] += jnp.sum(xb, axis=-1, keepdims=True)
    sq_ref[...] += jnp.sum(xb * xb, axis=-1, keepdims=True)


def _corr_kernel(x_ref, s1_ref, t1_ref, d_ref, hsum_ref, edge_ref, *, C, L):
    n = pl.program_id(0)

    @pl.when(n == 0)
    def _():
        d_ref[...] = jnp.zeros_like(d_ref)
        hsum_ref[...] = jnp.zeros_like(hsum_ref)

    h = jnp.maximum(x_ref[0] * s1_ref[...] + t1_ref[...], 0.0)
    hs = jnp.concatenate([h[:, 1:], jnp.zeros((C, 1), jnp.float32)], axis=1)
    hb = h.astype(jnp.bfloat16)
    rhs = jnp.concatenate([hb, hs.astype(jnp.bfloat16)], axis=0)
    d_ref[...] += jax.lax.dot_general(
        hb, rhs, (((1,), (1,)), ((), ())),
        preferred_element_type=jnp.float32)
    hsum_ref[...] += jnp.sum(h, axis=-1, keepdims=True)
    edge_ref[0] = jnp.concatenate([h[:, 0:1], h[:, L - 1:L]], axis=1)


def _out_kernel(x_ref, s1_ref, t1_ref, we_ref, wo_ref, t2_ref,
                o_ref, hb3_ref, *, C, L, TL):
    t = pl.program_id(1)

    @pl.when(t == 0)
    def _():
        s1 = s1_ref[...]
        t1 = t1_ref[...]
        CH = 256
        z1 = jnp.zeros((C, 1), jnp.float32)
        prev_last = z1
        prev_hc = None
        for m in range(L // CH):
            hc = jnp.maximum(x_ref[0, :, pl.ds(CH * m, CH)] * s1 + t1, 0.0)
            hb3_ref[C:2 * C, pl.ds(CH * m, CH)] = hc.astype(jnp.bfloat16)
            hm = jnp.concatenate([prev_last, hc[:, :CH - 1]], axis=1)
            hb3_ref[0:C, pl.ds(CH * m, CH)] = hm.astype(jnp.bfloat16)
            if prev_hc is not None:
                hp = jnp.concatenate([prev_hc[:, 1:], hc[:, 0:1]], axis=1)
                hb3_ref[2 * C:3 * C, pl.ds(CH * (m - 1), CH)] = (
                    hp.astype(jnp.bfloat16))
            prev_last = hc[:, CH - 1:CH]
            prev_hc = hc
        hp = jnp.concatenate([prev_hc[:, 1:], z1], axis=1)
        hb3_ref[2 * C:3 * C, pl.ds(L - CH, CH)] = hp.astype(jnp.bfloat16)

    t2 = t2_ref[...]
    F = t2.shape[0]
    iota = jax.lax.broadcasted_iota(jnp.int32, (F, 128), 1)
    idx_lo = iota >> 1
    idx_hi = 64 + (iota >> 1)
    even = (iota & 1) == 0
    for v in range(TL // 256):
        c0 = pl.multiple_of(t * TL + 256 * v, 128)
        sA = hb3_ref[0:2 * C, pl.ds(c0, 256)]
        sB = hb3_ref[C:3 * C, pl.ds(c0, 256)]
        yE = jnp.dot(we_ref[...], sA, preferred_element_type=jnp.float32)
        yO = jnp.dot(wo_ref[...], sB, preferred_element_type=jnp.float32)
        zE = jnp.maximum(yE + t2, 0.0)
        zO = jnp.maximum(yO + t2, 0.0)
        for u in range(2):
            Ev = zE[:, 128 * u:128 * u + 128]
            Ov = zO[:, 128 * u:128 * u + 128]
            lo_E = jnp.take_along_axis(Ev, idx_lo, axis=1)
            lo_O = jnp.take_along_axis(Ov, idx_lo, axis=1)
            o_ref[0, :, pl.ds(512 * v + 256 * u, 128)] = jnp.where(
                even, lo_E, lo_O)
            hi_E = jnp.take_along_axis(Ev, idx_hi, axis=1)
            hi_O = jnp.take_along_axis(Ov, idx_hi, axis=1)
            o_ref[0, :, pl.ds(512 * v + 256 * u + 128, 128)] = jnp.where(
                even, hi_E, hi_O)


def kernel(x, conv_w, conv_b, bn1_g, bn1_b, bn2_g, bn2_b):
    N, C, L = x.shape
    F = conv_w.shape[0]
    L2 = 2 * L
    TL = _pick_tile(L, 1024)
    T = L // TL

    x = x.astype(jnp.float32)
    conv_w = conv_w.astype(jnp.float32)
    conv_b = conv_b.astype(jnp.float32)
    bn1_g = bn1_g.astype(jnp.float32)
    bn1_b = bn1_b.astype(jnp.float32)
    bn2_g = bn2_g.astype(jnp.float32)
    bn2_b = bn2_b.astype(jnp.float32)

    vmem_limit = 64 * 1024 * 1024
    cp1 = pltpu.CompilerParams(dimension_semantics=("arbitrary",),
                               vmem_limit_bytes=vmem_limit)
    cp2 = pltpu.CompilerParams(dimension_semantics=("parallel", "arbitrary"),
                               vmem_limit_bytes=vmem_limit)

    xsum, xsq = pl.pallas_call(
        _xstats_kernel,
        out_shape=(jax.ShapeDtypeStruct((C, 1), jnp.float32),
                   jax.ShapeDtypeStruct((C, 1), jnp.float32)),
        grid=(N,),
        in_specs=[pl.BlockSpec((1, C, L), lambda n: (n, 0, 0))],
        out_specs=(pl.BlockSpec((C, 1), lambda n: (0, 0)),
                   pl.BlockSpec((C, 1), lambda n: (0, 0))),
        compiler_params=cp1,
    )(x)

    cnt1 = float(N * L)
    mean1 = xsum[:, 0] / cnt1
    var1 = xsq[:, 0] / cnt1 - mean1 ** 2
    sc1 = bn1_g * jax.lax.rsqrt(var1 + EPS)
    s1 = sc1.reshape(C, 1)
    t1 = (bn1_b - mean1 * sc1).reshape(C, 1)

    d, hsum, edge = pl.pallas_call(
        functools.partial(_corr_kernel, C=C, L=L),
        out_shape=(jax.ShapeDtypeStruct((C, 2 * C), jnp.float32),
                   jax.ShapeDtypeStruct((C, 1), jnp.float32),
                   jax.ShapeDtypeStruct((N, C, 2), jnp.float32)),
        grid=(N,),
        in_specs=[pl.BlockSpec((1, C, L), lambda n: (n, 0, 0)),
                  pl.BlockSpec((C, 1), lambda n: (0, 0)),
                  pl.BlockSpec((C, 1), lambda n: (0, 0))],
        out_specs=(pl.BlockSpec((C, 2 * C), lambda n: (0, 0)),
                   pl.BlockSpec((C, 1), lambda n: (0, 0)),
                   pl.BlockSpec((1, C, 2), lambda n: (n, 0, 0))),
        compiler_params=cp1,
    )(x, s1, t1)

    W0 = conv_w[:, :, 0]
    W1 = conv_w[:, :, 1]
    W2 = conv_w[:, :, 2]
    A = W0
    Bv = W1 + W2
    Cm = W0 + W1
    Dv = W2

    D0 = d[:, :C]
    D1 = d[:, C:]
    Sh = hsum[:, 0]
    H0 = edge[:, :, 0]
    HL = edge[:, :, 1]
    Q0 = H0.T @ H0
    QL = HL.T @ HL
    S0 = jnp.sum(H0, axis=0)
    SL = jnp.sum(HL, axis=0)

    U = jnp.concatenate([A, Bv], axis=1)
    V = jnp.concatenate([Cm, Dv], axis=1)
    ME = jnp.concatenate(
        [jnp.concatenate([D0 - QL, D1], axis=1),
         jnp.concatenate([D1.T, D0], axis=1)], axis=0)
    MO = jnp.concatenate(
        [jnp.concatenate([D0, D1], axis=1),
         jnp.concatenate([D1.T, D0 - Q0], axis=1)], axis=0)
    quad = (jnp.sum((U @ ME) * U, axis=1)
            + jnp.sum((V @ MO) * V, axis=1))
    lin = A @ (Sh - SL) + Bv @ Sh + Cm @ Sh + Dv @ (Sh - S0)
    cnt2 = float(N * L2)
    ysum = lin + cnt2 * conv_b
    ysq = quad + 2.0 * conv_b * lin + cnt2 * conv_b ** 2
    mean2 = ysum / cnt2
    var2 = ysq / cnt2 - mean2 ** 2
    sc2 = bn2_g * jax.lax.rsqrt(var2 + EPS)
    t2 = (bn2_b - mean2 * sc2 + sc2 * conv_b).reshape(F, 1)

    we = (U * sc2[:, None]).astype(jnp.bfloat16)
    wo = (V * sc2[:, None]).astype(jnp.bfloat16)

    z = pl.pallas_call(
        functools.partial(_out_kernel, C=C, L=L, TL=TL),
        out_shape=jax.ShapeDtypeStruct((N, F, L2), jnp.float32),
        grid=(N, T),
        in_specs=[pl.BlockSpec((1, C, L), lambda n, t: (n, 0, 0)),
                  pl.BlockSpec((C, 1), lambda n, t: (0, 0)),
                  pl.BlockSpec((C, 1), lambda n, t: (0, 0)),
                  pl.BlockSpec((F, 2 * C), lambda n, t: (0, 0)),
                  pl.BlockSpec((F, 2 * C), lambda n, t: (0, 0)),
                  pl.BlockSpec((F, 1), lambda n, t: (0, 0))],
        out_specs=pl.BlockSpec((1, F, 2 * TL), lambda n, t: (n, 0, t)),
        scratch_shapes=[pltpu.VMEM((3 * C, L), jnp.bfloat16)],
        compiler_params=cp2,
    )(x, s1, t1, we, wo, t2)
    return z


# speedup vs baseline: 3.6444x; 1.1719x over previous
import functools

import jax
import jax.numpy as jnp
from jax.experimental import pallas as pl
from jax.experimental.pallas import tpu as pltpu

EPS = 1e-5


def _pick_tile(n, target):
    if n <= target:
        return n
    t = (target // 128) * 128
    while t >= 128:
        if n % t == 0:
            return t
        t -= 128
    return n


def _xstats_kernel(x_ref, sum_ref, sq_ref):
    n = pl.program_id(0)

    @pl.when(n == 0)
    def _():
        sum_ref[...] = jnp.zeros_like(sum_ref)
        sq_ref[...] = jnp.zeros_like(sq_ref)

    xb = x_ref[0]
    sum_ref[...] += jnp.sum(xb, axis=-1, keepdims=True)
    sq_ref[...] += jnp.sum(xb * xb, axis=-1, keepdims=True)


def _corr_kernel(x_ref, s1_ref, t1_ref, d_ref, hsum_ref, edge_ref, *, C, L):
    n = pl.program_id(0)

    @pl.when(n == 0)
    def _():
        d_ref[...] = jnp.zeros_like(d_ref)
        hsum_ref[...] = jnp.zeros_like(hsum_ref)

    h = jnp.maximum(x_ref[0] * s1_ref[...] + t1_ref[...], 0.0)
    hs = jnp.concatenate([h[:, 1:], jnp.zeros((C, 1), jnp.float32)], axis=1)
    hb = h.astype(jnp.bfloat16)
    rhs = jnp.concatenate([hb, hs.astype(jnp.bfloat16)], axis=0)
    d_ref[...] += jax.lax.dot_general(
        hb, rhs, (((1,), (1,)), ((), ())),
        preferred_element_type=jnp.float32)
    hsum_ref[...] += jnp.sum(h, axis=-1, keepdims=True)
    edge_ref[0] = jnp.concatenate([h[:, 0:1], h[:, L - 1:L]], axis=1)


def _out_kernel(x_ref, s1_ref, t1_ref, we_ref, wo_ref, t2_ref,
                o_ref, hb3_ref, *, C, L, TL):
    t = pl.program_id(1)

    @pl.when(t == 0)
    def _():
        s1 = s1_ref[...]
        t1 = t1_ref[...]
        CH = 256
        z1 = jnp.zeros((C, 1), jnp.float32)
        prev_last = z1
        prev_hc = None
        for m in range(L // CH):
            hc = jnp.maximum(x_ref[0, :, pl.ds(CH * m, CH)] * s1 + t1, 0.0)
            hb3_ref[C:2 * C, pl.ds(CH * m, CH)] = hc.astype(jnp.bfloat16)
            hm = jnp.concatenate([prev_last, hc[:, :CH - 1]], axis=1)
            hb3_ref[0:C, pl.ds(CH * m, CH)] = hm.astype(jnp.bfloat16)
            if prev_hc is not None:
                hp = jnp.concatenate([prev_hc[:, 1:], hc[:, 0:1]], axis=1)
                hb3_ref[2 * C:3 * C, pl.ds(CH * (m - 1), CH)] = (
                    hp.astype(jnp.bfloat16))
            prev_last = hc[:, CH - 1:CH]
            prev_hc = hc
        hp = jnp.concatenate([prev_hc[:, 1:], z1], axis=1)
        hb3_ref[2 * C:3 * C, pl.ds(L - CH, CH)] = hp.astype(jnp.bfloat16)

    t2 = t2_ref[...]
    F = t2.shape[0]
    iota = jax.lax.broadcasted_iota(jnp.int32, (F, 128), 1)
    idx_lo = iota >> 1
    idx_hi = 64 + (iota >> 1)
    even = (iota & 1) == 0
    for v in range(TL // 256):
        c0 = pl.multiple_of(t * TL + 256 * v, 128)
        sA = hb3_ref[0:2 * C, pl.ds(c0, 256)]
        sB = hb3_ref[C:3 * C, pl.ds(c0, 256)]
        yE = jnp.dot(we_ref[...], sA, preferred_element_type=jnp.float32)
        yO = jnp.dot(wo_ref[...], sB, preferred_element_type=jnp.float32)
        zE = jnp.maximum(yE + t2, 0.0)
        zO = jnp.maximum(yO + t2, 0.0)
        for u in range(2):
            Ev = zE[:, 128 * u:128 * u + 128]
            Ov = zO[:, 128 * u:128 * u + 128]
            lo_E = jnp.take_along_axis(Ev, idx_lo, axis=1)
            lo_O = jnp.take_along_axis(Ov, idx_lo, axis=1)
            o_ref[0, :, pl.ds(512 * v + 256 * u, 128)] = jnp.where(
                even, lo_E, lo_O)
            hi_E = jnp.take_along_axis(Ev, idx_hi, axis=1)
            hi_O = jnp.take_along_axis(Ov, idx_hi, axis=1)
            o_ref[0, :, pl.ds(512 * v + 256 * u + 128, 128)] = jnp.where(
                even, hi_E, hi_O)


def kernel(x, conv_w, conv_b, bn1_g, bn1_b, bn2_g, bn2_b):
    N, C, L = x.shape
    F = conv_w.shape[0]
    L2 = 2 * L
    TL = _pick_tile(L, 2048)
    T = L // TL

    x = x.astype(jnp.float32)
    conv_w = conv_w.astype(jnp.float32)
    conv_b = conv_b.astype(jnp.float32)
    bn1_g = bn1_g.astype(jnp.float32)
    bn1_b = bn1_b.astype(jnp.float32)
    bn2_g = bn2_g.astype(jnp.float32)
    bn2_b = bn2_b.astype(jnp.float32)

    vmem_limit = 64 * 1024 * 1024
    cp1 = pltpu.CompilerParams(dimension_semantics=("arbitrary",),
                               vmem_limit_bytes=vmem_limit)
    cp2 = pltpu.CompilerParams(dimension_semantics=("parallel", "arbitrary"),
                               vmem_limit_bytes=vmem_limit)

    xsum, xsq = pl.pallas_call(
        _xstats_kernel,
        out_shape=(jax.ShapeDtypeStruct((C, 1), jnp.float32),
                   jax.ShapeDtypeStruct((C, 1), jnp.float32)),
        grid=(N,),
        in_specs=[pl.BlockSpec((1, C, L), lambda n: (n, 0, 0))],
        out_specs=(pl.BlockSpec((C, 1), lambda n: (0, 0)),
                   pl.BlockSpec((C, 1), lambda n: (0, 0))),
        compiler_params=cp1,
    )(x)

    cnt1 = float(N * L)
    mean1 = xsum[:, 0] / cnt1
    var1 = xsq[:, 0] / cnt1 - mean1 ** 2
    sc1 = bn1_g * jax.lax.rsqrt(var1 + EPS)
    s1 = sc1.reshape(C, 1)
    t1 = (bn1_b - mean1 * sc1).reshape(C, 1)

    d, hsum, edge = pl.pallas_call(
        functools.partial(_corr_kernel, C=C, L=L),
        out_shape=(jax.ShapeDtypeStruct((C, 2 * C), jnp.float32),
                   jax.ShapeDtypeStruct((C, 1), jnp.float32),
                   jax.ShapeDtypeStruct((N, C, 2), jnp.float32)),
        grid=(N,),
        in_specs=[pl.BlockSpec((1, C, L), lambda n: (n, 0, 0)),
                  pl.BlockSpec((C, 1), lambda n: (0, 0)),
                  pl.BlockSpec((C, 1), lambda n: (0, 0))],
        out_specs=(pl.BlockSpec((C, 2 * C), lambda n: (0, 0)),
                   pl.BlockSpec((C, 1), lambda n: (0, 0)),
                   pl.BlockSpec((1, C, 2), lambda n: (n, 0, 0))),
        compiler_params=cp1,
    )(x, s1, t1)

    W0 = conv_w[:, :, 0]
    W1 = conv_w[:, :, 1]
    W2 = conv_w[:, :, 2]
    A = W0
    Bv = W1 + W2
    Cm = W0 + W1
    Dv = W2

    D0 = d[:, :C]
    D1 = d[:, C:]
    Sh = hsum[:, 0]
    H0 = edge[:, :, 0]
    HL = edge[:, :, 1]
    Q0 = H0.T @ H0
    QL = HL.T @ HL
    S0 = jnp.sum(H0, axis=0)
    SL = jnp.sum(HL, axis=0)

    U = jnp.concatenate([A, Bv], axis=1)
    V = jnp.concatenate([Cm, Dv], axis=1)
    ME = jnp.concatenate(
        [jnp.concatenate([D0 - QL, D1], axis=1),
         jnp.concatenate([D1.T, D0], axis=1)], axis=0)
    MO = jnp.concatenate(
        [jnp.concatenate([D0, D1], axis=1),
         jnp.concatenate([D1.T, D0 - Q0], axis=1)], axis=0)
    quad = (jnp.sum((U @ ME) * U, axis=1)
            + jnp.sum((V @ MO) * V, axis=1))
    lin = A @ (Sh - SL) + Bv @ Sh + Cm @ Sh + Dv @ (Sh - S0)
    cnt2 = float(N * L2)
    ysum = lin + cnt2 * conv_b
    ysq = quad + 2.0 * conv_b * lin + cnt2 * conv_b ** 2
    mean2 = ysum / cnt2
    var2 = ysq / cnt2 - mean2 ** 2
    sc2 = bn2_g * jax.lax.rsqrt(var2 + EPS)
    t2 = (bn2_b - mean2 * sc2 + sc2 * conv_b).reshape(F, 1)

    we = (U * sc2[:, None]).astype(jnp.bfloat16)
    wo = (V * sc2[:, None]).astype(jnp.bfloat16)

    z = pl.pallas_call(
        functools.partial(_out_kernel, C=C, L=L, TL=TL),
        out_shape=jax.ShapeDtypeStruct((N, F, L2), jnp.float32),
        grid=(N, T),
        in_specs=[pl.BlockSpec((1, C, L), lambda n, t: (n, 0, 0)),
                  pl.BlockSpec((C, 1), lambda n, t: (0, 0)),
                  pl.BlockSpec((C, 1), lambda n, t: (0, 0)),
                  pl.BlockSpec((F, 2 * C), lambda n, t: (0, 0)),
                  pl.BlockSpec((F, 2 * C), lambda n, t: (0, 0)),
                  pl.BlockSpec((F, 1), lambda n, t: (0, 0))],
        out_specs=pl.BlockSpec((1, F, 2 * TL), lambda n, t: (n, 0, t)),
        scratch_shapes=[pltpu.VMEM((3 * C, L), jnp.bfloat16)],
        compiler_params=cp2,
    )(x, s1, t1, we, wo, t2)
    return z
